# Initial kernel scaffold; baseline (speedup 1.0000x reference)
import jax
import jax.numpy as jnp
from jax.experimental import pallas as pl


def kernel(x, edge_index, edge_attr, Wq, bq, Wk, bk, Wv, bv, We, be, Wskip, bskip, g1, be1, W1, bf1, W2, bf2, g2, be2):
    raise NotImplementedError("write your pallas kernel here")


# speedup vs baseline: 24.2002x; 24.2002x over previous
import functools
import math

import jax
import jax.numpy as jnp
from jax import lax
from jax.experimental import pallas as pl
from jax.experimental.pallas import tpu as pltpu
from jax.experimental.pallas import tpu_sc as plsc

N = 10000
E = 320000
D = 128
H = 8
Dh = 16
KV = 2 * D
PW = 144

NC = 2
NS = 16
NW = NC * NS
EPW = E // NW
CH = 128
NFULL = EPW // CH
TAIL = EPW - NFULL * CH
NP = 10240
RPT = NP // NS
ZR = 64
SCH = 64
SNFULL = EPW // SCH
STAIL = EPW - SNFULL * SCH

_f32 = jnp.float32


def _proj_body(x_ref, w_ref, b_ref, o_ref):
    o_ref[...] = (
        jnp.dot(x_ref[...], w_ref[...], preferred_element_type=_f32)
        + b_ref[...]
    )


def _edge_body(qd_ref, kvs_ref, ea_ref, we_ref, be_ref, s_ref, st_ref, o_ref):
    e = jnp.dot(ea_ref[...], we_ref[...], preferred_element_type=_f32) + be_ref[...]
    k_e = kvs_ref[:, :D] + e
    v_e = kvs_ref[:, D:] + e
    prod = qd_ref[...] * k_e
    logits = jnp.dot(prod, s_ref[...], preferred_element_type=_f32) * (
        1.0 / math.sqrt(Dh)
    )
    ex = jnp.exp(logits)
    ex_b = jnp.dot(ex, st_ref[...], preferred_element_type=_f32)
    wv = ex_b * v_e
    pad = jnp.zeros(ex.shape, _f32)
    o_ref[...] = jnp.concatenate([wv, ex, pad], axis=-1)


def _layernorm(x, g, b, eps=1e-5):
    mu = jnp.mean(x, axis=-1, keepdims=True)
    var = jnp.mean((x - mu) ** 2, axis=-1, keepdims=True)
    return (x - mu) / jnp.sqrt(var + eps) * g + b


def _final_body(p0_ref, p1_ref, x_ref, xs_ref, w1_ref, b1_ref, w2_ref,
                b2_ref, g1_ref, be1_ref, g2_ref, be2_ref, st_ref, o_ref):
    numer = p0_ref[0, :, :D] + p1_ref[0, :, :D]
    den = p0_ref[0, :, D:D + H] + p1_ref[0, :, D:D + H]
    den_b = jnp.dot(jnp.maximum(den, 1e-16), st_ref[...],
                    preferred_element_type=_f32)
    agg = numer / den_b
    conv = agg + xs_ref[...] + x_ref[...]
    h = _layernorm(conv, g1_ref[...], be1_ref[...])
    t = jnp.dot(h, w1_ref[...], preferred_element_type=_f32) + b1_ref[...]
    ff = jnp.dot(jax.nn.gelu(t), w2_ref[...], preferred_element_type=_f32) \
        + b2_ref[...]
    o_ref[...] = _layernorm(ff + h, g2_ref[...], be2_ref[...])


def _sc_gather_body(kv_hbm, q_hbm, src_hbm, dst_hbm, kvs_out, qd_out,
               si, di, kvbuf, qbuf, si_t, di_t, kvbuf_t, qbuf_t, sem_k, sem_q):
    c = lax.axis_index("c")
    s = lax.axis_index("s")
    base = (c * NS + s) * EPW

    def chunk(off, idx_s, idx_d, buf_kv, buf_q, n):
        pltpu.sync_copy(src_hbm.at[pl.ds(off, n)], idx_s)
        pltpu.sync_copy(dst_hbm.at[pl.ds(off, n)], idx_d)
        ck = pltpu.async_copy(kv_hbm.at[idx_s], buf_kv, sem_k)
        cq = pltpu.async_copy(q_hbm.at[idx_d], buf_q, sem_q)
        ck.wait()
        cq.wait()
        pltpu.sync_copy(buf_kv, kvs_out.at[pl.ds(off, n)])
        pltpu.sync_copy(buf_q, qd_out.at[pl.ds(off, n)])

    def body(i, carry):
        chunk(base + i * CH, si, di, kvbuf, qbuf, CH)
        return carry

    lax.fori_loop(0, NFULL, body, 0)
    chunk(base + NFULL * CH, si_t, di_t, kvbuf_t, qbuf_t, TAIL)


def _sc_scatter_body(wvx_hbm, dst_hbm, out_hbm, acc, di, wbuf, di_t, wbuf_t,
                     zbuf):
    c = lax.axis_index("c")
    s = lax.axis_index("s")
    base = (c * NS + s) * EPW

    def zrow(i, carry):
        for j in range(PW // 16):
            zbuf[i, pl.ds(j * 16, 16)] = jnp.zeros((16,), _f32)
        return carry

    lax.fori_loop(0, ZR, zrow, 0)

    def zcopy(j, carry):
        pltpu.sync_copy(zbuf, acc.at[pl.ds(s * RPT + j * ZR, ZR)])
        return carry

    lax.fori_loop(0, RPT // ZR, zcopy, 0)
    plsc.subcore_barrier()

    def chunk(off, idx, buf, n):
        pltpu.sync_copy(dst_hbm.at[pl.ds(off, n)], idx)
        pltpu.sync_copy(wvx_hbm.at[pl.ds(off, n)], buf)
        pltpu.sync_copy(buf, acc.at[idx], add=True)

    def body(i, carry):
        chunk(base + i * SCH, di, wbuf, SCH)
        return carry

    lax.fori_loop(0, SNFULL, body, 0)
    chunk(base + SNFULL * SCH, di_t, wbuf_t, STAIL)

    plsc.subcore_barrier()
    pltpu.sync_copy(acc.at[pl.ds(s * RPT, RPT)],
                    out_hbm.at[c, pl.ds(s * RPT, RPT)])


@functools.lru_cache(maxsize=1)
def _sc_kernels():
    mesh = plsc.VectorSubcoreMesh(core_axis_name="c", subcore_axis_name="s")
    gather = functools.partial(
        pl.kernel,
        out_type=(
            jax.ShapeDtypeStruct((E, KV), _f32),
            jax.ShapeDtypeStruct((E, D), _f32),
        ),
        mesh=mesh,
        scratch_types=[
            pltpu.VMEM((CH,), jnp.int32),
            pltpu.VMEM((CH,), jnp.int32),
            pltpu.VMEM((CH, KV), _f32),
            pltpu.VMEM((CH, D), _f32),
            pltpu.VMEM((TAIL,), jnp.int32),
            pltpu.VMEM((TAIL,), jnp.int32),
            pltpu.VMEM((TAIL, KV), _f32),
            pltpu.VMEM((TAIL, D), _f32),
            pltpu.SemaphoreType.DMA,
            pltpu.SemaphoreType.DMA,
        ],
    )(_sc_gather_body)
    scatter = functools.partial(
        pl.kernel,
        out_type=jax.ShapeDtypeStruct((NC, NP, PW), _f32),
        mesh=mesh,
        scratch_types=[
            pltpu.VMEM_SHARED((NP, PW), _f32),
            pltpu.VMEM((SCH,), jnp.int32),
            pltpu.VMEM((SCH, PW), _f32),
            pltpu.VMEM((STAIL,), jnp.int32),
            pltpu.VMEM((STAIL, PW), _f32),
            pltpu.VMEM((ZR, PW), _f32),
        ],
        compiler_params=pltpu.CompilerParams(use_tc_tiling_on_sc=False),
    )(_sc_scatter_body)
    return gather, scatter


_BN = 1000
_BE = 512


def _proj(x, wcat, bcat):
    return pl.pallas_call(
        _proj_body,
        grid=(N // _BN,),
        in_specs=[
            pl.BlockSpec((_BN, D), lambda i: (i, 0)),
            pl.BlockSpec((D, 4 * D), lambda i: (0, 0)),
            pl.BlockSpec((1, 4 * D), lambda i: (0, 0)),
        ],
        out_specs=pl.BlockSpec((_BN, 4 * D), lambda i: (i, 0)),
        out_shape=jax.ShapeDtypeStruct((N, 4 * D), _f32),
    )(x, wcat, bcat)


def _edge(qd, kvs, ea, we, be, s_mat, st_mat):
    return pl.pallas_call(
        _edge_body,
        grid=(E // _BE,),
        in_specs=[
            pl.BlockSpec((_BE, D), lambda i: (i, 0)),
            pl.BlockSpec((_BE, KV), lambda i: (i, 0)),
            pl.BlockSpec((_BE, 16), lambda i: (i, 0)),
            pl.BlockSpec((16, D), lambda i: (0, 0)),
            pl.BlockSpec((1, D), lambda i: (0, 0)),
            pl.BlockSpec((D, H), lambda i: (0, 0)),
            pl.BlockSpec((H, D), lambda i: (0, 0)),
        ],
        out_specs=pl.BlockSpec((_BE, PW), lambda i: (i, 0)),
        out_shape=jax.ShapeDtypeStruct((E, PW), _f32),
    )(qd, kvs, ea, we, be, s_mat, st_mat)


def _final(parts, x, xs, w1, b1, w2, b2, g1, be1, g2, be2, st_mat):
    return pl.pallas_call(
        _final_body,
        grid=(N // _BN,),
        in_specs=[
            pl.BlockSpec((1, _BN, PW), lambda i: (0, i, 0)),
            pl.BlockSpec((1, _BN, PW), lambda i: (1, i, 0)),
            pl.BlockSpec((_BN, D), lambda i: (i, 0)),
            pl.BlockSpec((_BN, D), lambda i: (i, 0)),
            pl.BlockSpec((D, 4 * D), lambda i: (0, 0)),
            pl.BlockSpec((1, 4 * D), lambda i: (0, 0)),
            pl.BlockSpec((4 * D, D), lambda i: (0, 0)),
            pl.BlockSpec((1, D), lambda i: (0, 0)),
            pl.BlockSpec((1, D), lambda i: (0, 0)),
            pl.BlockSpec((1, D), lambda i: (0, 0)),
            pl.BlockSpec((1, D), lambda i: (0, 0)),
            pl.BlockSpec((1, D), lambda i: (0, 0)),
            pl.BlockSpec((H, D), lambda i: (0, 0)),
        ],
        out_specs=pl.BlockSpec((_BN, D), lambda i: (i, 0)),
        out_shape=jax.ShapeDtypeStruct((N, D), _f32),
    )(parts, parts, x, xs, w1, b1, w2, b2, g1, be1, g2, be2, st_mat)


def kernel(x, edge_index, edge_attr, Wq, bq, Wk, bk, Wv, bv, We, be,
           Wskip, bskip, g1, be1, W1, bf1, W2, bf2, g2, be2):
    src = edge_index[0]
    dst = edge_index[1]

    wcat = jnp.concatenate([Wk, Wv, Wq, Wskip], axis=1)
    bcat = jnp.concatenate([bk, bv, bq, bskip])[None, :]
    proj = _proj(x, wcat, bcat)
    kv_tab = proj[:, :KV]
    q_tab = proj[:, KV:KV + D]
    xs = proj[:, KV + D:]

    sc_gather, sc_scatter = _sc_kernels()
    kvs, qd = sc_gather(kv_tab, q_tab, src, dst)

    s_mat = jnp.kron(jnp.eye(H, dtype=_f32), jnp.ones((Dh, 1), _f32))
    st_mat = s_mat.T
    wvx = _edge(qd, kvs, edge_attr, We, be[None, :], s_mat, st_mat)

    parts = sc_scatter(wvx, dst)

    return _final(parts, x, xs, W1, bf1[None, :], W2, bf2[None, :],
                  g1[None, :], be1[None, :], g2[None, :], be2[None, :], st_mat)


# speedup vs baseline: 29.3625x; 1.2133x over previous
import functools
import math

import jax
import jax.numpy as jnp
from jax import lax
from jax.experimental import pallas as pl
from jax.experimental.pallas import tpu as pltpu
from jax.experimental.pallas import tpu_sc as plsc

N = 10000
E = 320000
D = 128
H = 8
Dh = 16
KV = 2 * D
PW = 144

NCH = 5
EC = E // NCH

NC = 2
NS = 16
NW = NC * NS
EPW = EC // NW
CH = 128
NFULL = EPW // CH
TAIL = EPW - NFULL * CH
NP = 10240
RPT = NP // NS
ZR = 64
SCH = 64
SNFULL = EPW // SCH
STAIL = EPW - SNFULL * SCH

_f32 = jnp.float32


def _proj_body(x_ref, w_ref, b_ref, o_ref):
    o_ref[...] = (
        jnp.dot(x_ref[...], w_ref[...], preferred_element_type=_f32)
        + b_ref[...]
    )


def _edge_body(qd_ref, kvs_ref, ea_ref, we_ref, be_ref, s_ref, st_ref, o_ref):
    e = jnp.dot(ea_ref[...], we_ref[...], preferred_element_type=_f32) + be_ref[...]
    k_e = kvs_ref[:, :D] + e
    v_e = kvs_ref[:, D:] + e
    prod = qd_ref[...] * k_e
    logits = jnp.dot(prod, s_ref[...], preferred_element_type=_f32)
    ex = jnp.exp(logits)
    ex_b = jnp.dot(ex, st_ref[...], preferred_element_type=_f32)
    o_ref[:, :D] = ex_b * v_e
    o_ref[:, D:D + H] = ex
    o_ref[:, D + H:] = jnp.zeros((ex.shape[0], PW - D - H), _f32)


def _layernorm(x, g, b, eps=1e-5):
    mu = jnp.mean(x, axis=-1, keepdims=True)
    var = jnp.mean((x - mu) ** 2, axis=-1, keepdims=True)
    return (x - mu) / jnp.sqrt(var + eps) * g + b


def _final_body(p0_ref, p1_ref, p2_ref, p3_ref, p4_ref, x_ref, xs_ref, w1_ref,
                b1_ref, w2_ref, b2_ref, g1_ref, be1_ref, g2_ref, be2_ref,
                st_ref, o_ref):
    acc = ((p0_ref[0] + p0_ref[1]) + (p1_ref[0] + p1_ref[1])
           + (p2_ref[0] + p2_ref[1]) + (p3_ref[0] + p3_ref[1])
           + (p4_ref[0] + p4_ref[1]))
    numer = acc[:, :D]
    den = acc[:, D:D + H]
    den_b = jnp.dot(jnp.maximum(den, 1e-16), st_ref[...],
                    preferred_element_type=_f32)
    agg = numer / den_b
    conv = agg + xs_ref[...] + x_ref[...]
    h = _layernorm(conv, g1_ref[...], be1_ref[...])
    t = jnp.dot(h, w1_ref[...], preferred_element_type=_f32) + b1_ref[...]
    ff = jnp.dot(jax.nn.gelu(t), w2_ref[...], preferred_element_type=_f32) \
        + b2_ref[...]
    o_ref[...] = _layernorm(ff + h, g2_ref[...], be2_ref[...])


def _sc_gather_body(kv_hbm, q_hbm, src_hbm, dst_hbm, kvs_out, qd_out,
               si, di, kvbuf, qbuf, si_t, di_t, kvbuf_t, qbuf_t, sem_k, sem_q):
    c = lax.axis_index("c")
    s = lax.axis_index("s")
    base = (c * NS + s) * EPW

    def chunk(off, idx_s, idx_d, buf_kv, buf_q, n):
        pltpu.sync_copy(src_hbm.at[pl.ds(off, n)], idx_s)
        pltpu.sync_copy(dst_hbm.at[pl.ds(off, n)], idx_d)
        ck = pltpu.async_copy(kv_hbm.at[idx_s], buf_kv, sem_k)
        cq = pltpu.async_copy(q_hbm.at[idx_d], buf_q, sem_q)
        ck.wait()
        cq.wait()
        pltpu.sync_copy(buf_kv, kvs_out.at[pl.ds(off, n)])
        pltpu.sync_copy(buf_q, qd_out.at[pl.ds(off, n)])

    def body(i, carry):
        chunk(base + i * CH, si, di, kvbuf, qbuf, CH)
        return carry

    lax.fori_loop(0, NFULL, body, 0)
    chunk(base + NFULL * CH, si_t, di_t, kvbuf_t, qbuf_t, TAIL)


def _sc_scatter_body(wvx_hbm, dst_hbm, out_hbm, acc, di, wbuf, di_t, wbuf_t,
                     zbuf):
    c = lax.axis_index("c")
    s = lax.axis_index("s")
    base = (c * NS + s) * EPW

    def zrow(i, carry):
        for j in range(PW // 16):
            zbuf[i, pl.ds(j * 16, 16)] = jnp.zeros((16,), _f32)
        return carry

    lax.fori_loop(0, ZR, zrow, 0)

    def zcopy(j, carry):
        pltpu.sync_copy(zbuf, acc.at[pl.ds(s * RPT + j * ZR, ZR)])
        return carry

    lax.fori_loop(0, RPT // ZR, zcopy, 0)
    plsc.subcore_barrier()

    def chunk(off, idx, buf, n):
        pltpu.sync_copy(dst_hbm.at[pl.ds(off, n)], idx)
        pltpu.sync_copy(wvx_hbm.at[pl.ds(off, n)], buf)
        pltpu.sync_copy(buf, acc.at[idx], add=True)

    def body(i, carry):
        chunk(base + i * SCH, di, wbuf, SCH)
        return carry

    lax.fori_loop(0, SNFULL, body, 0)
    chunk(base + SNFULL * SCH, di_t, wbuf_t, STAIL)

    plsc.subcore_barrier()
    pltpu.sync_copy(acc.at[pl.ds(s * RPT, RPT)],
                    out_hbm.at[c, pl.ds(s * RPT, RPT)])


@functools.lru_cache(maxsize=1)
def _sc_kernels():
    mesh = plsc.VectorSubcoreMesh(core_axis_name="c", subcore_axis_name="s")
    gather = functools.partial(
        pl.kernel,
        out_type=(
            jax.ShapeDtypeStruct((EC, KV), _f32),
            jax.ShapeDtypeStruct((EC, D), _f32),
        ),
        mesh=mesh,
        scratch_types=[
            pltpu.VMEM((CH,), jnp.int32),
            pltpu.VMEM((CH,), jnp.int32),
            pltpu.VMEM((CH, KV), _f32),
            pltpu.VMEM((CH, D), _f32),
            pltpu.VMEM((TAIL,), jnp.int32),
            pltpu.VMEM((TAIL,), jnp.int32),
            pltpu.VMEM((TAIL, KV), _f32),
            pltpu.VMEM((TAIL, D), _f32),
            pltpu.SemaphoreType.DMA,
            pltpu.SemaphoreType.DMA,
        ],
    )(_sc_gather_body)
    scatter = functools.partial(
        pl.kernel,
        out_type=jax.ShapeDtypeStruct((NC, NP, PW), _f32),
        mesh=mesh,
        scratch_types=[
            pltpu.VMEM_SHARED((NP, PW), _f32),
            pltpu.VMEM((SCH,), jnp.int32),
            pltpu.VMEM((SCH, PW), _f32),
            pltpu.VMEM((STAIL,), jnp.int32),
            pltpu.VMEM((STAIL, PW), _f32),
            pltpu.VMEM((ZR, PW), _f32),
        ],
        compiler_params=pltpu.CompilerParams(use_tc_tiling_on_sc=False),
    )(_sc_scatter_body)
    return gather, scatter


_BN = 1000
_BE = 512


def _proj(x, wcat, bcat):
    return pl.pallas_call(
        _proj_body,
        grid=(N // _BN,),
        in_specs=[
            pl.BlockSpec((_BN, D), lambda i: (i, 0)),
            pl.BlockSpec((D, 4 * D), lambda i: (0, 0)),
            pl.BlockSpec((1, 4 * D), lambda i: (0, 0)),
        ],
        out_specs=pl.BlockSpec((_BN, 4 * D), lambda i: (i, 0)),
        out_shape=jax.ShapeDtypeStruct((N, 4 * D), _f32),
    )(x, wcat, bcat)


def _edge(qd, kvs, ea, we, be, s_mat, st_mat):
    return pl.pallas_call(
        _edge_body,
        grid=(EC // _BE,),
        in_specs=[
            pl.BlockSpec((_BE, D), lambda i: (i, 0)),
            pl.BlockSpec((_BE, KV), lambda i: (i, 0)),
            pl.BlockSpec((_BE, 16), lambda i: (i, 0)),
            pl.BlockSpec((16, D), lambda i: (0, 0)),
            pl.BlockSpec((1, D), lambda i: (0, 0)),
            pl.BlockSpec((D, H), lambda i: (0, 0)),
            pl.BlockSpec((H, D), lambda i: (0, 0)),
        ],
        out_specs=pl.BlockSpec((_BE, PW), lambda i: (i, 0)),
        out_shape=jax.ShapeDtypeStruct((EC, PW), _f32),
    )(qd, kvs, ea, we, be, s_mat, st_mat)


def _final(parts, x, xs, w1, b1, w2, b2, g1, be1, g2, be2, st_mat):
    return pl.pallas_call(
        _final_body,
        grid=(N // _BN,),
        in_specs=[
            pl.BlockSpec((NC, _BN, PW), lambda i: (0, i, 0)),
            pl.BlockSpec((NC, _BN, PW), lambda i: (0, i, 0)),
            pl.BlockSpec((NC, _BN, PW), lambda i: (0, i, 0)),
            pl.BlockSpec((NC, _BN, PW), lambda i: (0, i, 0)),
            pl.BlockSpec((NC, _BN, PW), lambda i: (0, i, 0)),
            pl.BlockSpec((_BN, D), lambda i: (i, 0)),
            pl.BlockSpec((_BN, D), lambda i: (i, 0)),
            pl.BlockSpec((D, 4 * D), lambda i: (0, 0)),
            pl.BlockSpec((1, 4 * D), lambda i: (0, 0)),
            pl.BlockSpec((4 * D, D), lambda i: (0, 0)),
            pl.BlockSpec((1, D), lambda i: (0, 0)),
            pl.BlockSpec((1, D), lambda i: (0, 0)),
            pl.BlockSpec((1, D), lambda i: (0, 0)),
            pl.BlockSpec((1, D), lambda i: (0, 0)),
            pl.BlockSpec((1, D), lambda i: (0, 0)),
            pl.BlockSpec((H, D), lambda i: (0, 0)),
        ],
        out_specs=pl.BlockSpec((_BN, D), lambda i: (i, 0)),
        out_shape=jax.ShapeDtypeStruct((N, D), _f32),
    )(*parts, x, xs, w1, b1, w2, b2, g1, be1, g2, be2, st_mat)


def kernel(x, edge_index, edge_attr, Wq, bq, Wk, bk, Wv, bv, We, be,
           Wskip, bskip, g1, be1, W1, bf1, W2, bf2, g2, be2):
    src = edge_index[0]
    dst = edge_index[1]

    wcat = jnp.concatenate([Wk, Wv, Wq, Wskip], axis=1)
    bcat = jnp.concatenate([bk, bv, bq, bskip])[None, :]
    proj = _proj(x, wcat, bcat)
    kv_tab = proj[:, :KV]
    q_tab = proj[:, KV:KV + D]
    xs = proj[:, KV + D:]

    sc_gather, sc_scatter = _sc_kernels()

    hs = jnp.kron(jnp.eye(H, dtype=_f32), jnp.ones((Dh, 1), _f32))
    s_mat = hs * (1.0 / math.sqrt(Dh))
    st_mat = hs.T

    parts = []
    for i in range(NCH):
        sl = slice(i * EC, (i + 1) * EC)
        kvs, qd = sc_gather(kv_tab, q_tab, src[sl], dst[sl])
        wvx = _edge(qd, kvs, edge_attr[sl], We, be[None, :], s_mat, st_mat)
        parts.append(sc_scatter(wvx, dst[sl]))

    return _final(parts, x, xs, W1, bf1[None, :], W2, bf2[None, :],
                  g1[None, :], be1[None, :], g2[None, :], be2[None, :], st_mat)


# speedup vs baseline: 35.2779x; 1.2015x over previous
import functools
import math

import jax
import jax.numpy as jnp
from jax import lax
from jax.experimental import pallas as pl
from jax.experimental.pallas import tpu as pltpu
from jax.experimental.pallas import tpu_sc as plsc

N = 10000
E = 320000
D = 128
H = 8
Dh = 16
KV = 2 * D
EW = 16

NCH = 5
EC = E // NCH

NC = 2
NS = 16
NW = NC * NS
EPW = EC // NW
CH = 128
NFULL = EPW // CH
TAIL = EPW - NFULL * CH
NP = 10240
RPT = NP // NS
ZR = 64
SCH = 64
SNFULL = EPW // SCH
STAIL = EPW - SNFULL * SCH

_f32 = jnp.float32


def _proj_body(x_ref, w_ref, b_ref, o_ref):
    o_ref[...] = (
        jnp.dot(x_ref[...], w_ref[...], preferred_element_type=_f32)
        + b_ref[...]
    )


def _edge_body(qd_ref, kvs_ref, ea_ref, we_ref, be_ref, s_ref, st_ref,
               wv_ref, ex_ref):
    e = jnp.dot(ea_ref[...], we_ref[...], preferred_element_type=_f32) + be_ref[...]
    k_e = kvs_ref[:, :D] + e
    v_e = kvs_ref[:, D:] + e
    prod = qd_ref[...] * k_e
    logits = jnp.dot(prod, s_ref[...], preferred_element_type=_f32)
    ex = jnp.exp(logits)
    ex_b = jnp.dot(ex, st_ref[...], preferred_element_type=_f32)
    wv_ref[...] = ex_b * v_e
    ex_ref[:, :H] = ex
    ex_ref[:, H:] = jnp.zeros((ex.shape[0], EW - H), _f32)


def _layernorm(x, g, b, eps=1e-5):
    mu = jnp.mean(x, axis=-1, keepdims=True)
    var = jnp.mean((x - mu) ** 2, axis=-1, keepdims=True)
    return (x - mu) / jnp.sqrt(var + eps) * g + b


def _final_body(pw0, pe0, pw1, pe1, pw2, pe2, pw3, pe3, pw4, pe4, x_ref,
                xs_ref, w1_ref, b1_ref, w2_ref, b2_ref, g1_ref, be1_ref,
                g2_ref, be2_ref, st_ref, o_ref):
    numer = ((pw0[0] + pw0[1]) + (pw1[0] + pw1[1]) + (pw2[0] + pw2[1])
             + (pw3[0] + pw3[1]) + (pw4[0] + pw4[1]))
    eacc = ((pe0[0] + pe0[1]) + (pe1[0] + pe1[1]) + (pe2[0] + pe2[1])
            + (pe3[0] + pe3[1]) + (pe4[0] + pe4[1]))
    den = eacc[:, :H]
    den_b = jnp.dot(jnp.maximum(den, 1e-16), st_ref[...],
                    preferred_element_type=_f32)
    agg = numer / den_b
    conv = agg + xs_ref[...] + x_ref[...]
    h = _layernorm(conv, g1_ref[...], be1_ref[...])
    t = jnp.dot(h, w1_ref[...], preferred_element_type=_f32) + b1_ref[...]
    ff = jnp.dot(jax.nn.gelu(t), w2_ref[...], preferred_element_type=_f32) \
        + b2_ref[...]
    o_ref[...] = _layernorm(ff + h, g2_ref[...], be2_ref[...])


def _sc_gather_body(kv_hbm, q_hbm, src_hbm, dst_hbm, kvs_out, qd_out,
               si, di, kvbuf, qbuf, si_t, di_t, kvbuf_t, qbuf_t, sem_k, sem_q):
    c = lax.axis_index("c")
    s = lax.axis_index("s")
    base = (c * NS + s) * EPW

    def chunk(off, idx_s, idx_d, buf_kv, buf_q, n):
        pltpu.sync_copy(src_hbm.at[pl.ds(off, n)], idx_s)
        pltpu.sync_copy(dst_hbm.at[pl.ds(off, n)], idx_d)
        ck = pltpu.async_copy(kv_hbm.at[idx_s], buf_kv, sem_k)
        cq = pltpu.async_copy(q_hbm.at[idx_d], buf_q, sem_q)
        ck.wait()
        cq.wait()
        pltpu.sync_copy(buf_kv, kvs_out.at[pl.ds(off, n)])
        pltpu.sync_copy(buf_q, qd_out.at[pl.ds(off, n)])

    def body(i, carry):
        chunk(base + i * CH, si, di, kvbuf, qbuf, CH)
        return carry

    lax.fori_loop(0, NFULL, body, 0)
    chunk(base + NFULL * CH, si_t, di_t, kvbuf_t, qbuf_t, TAIL)


def _sc_scatter_body(wv_hbm, ex_hbm, dst_hbm, outw_hbm, oute_hbm, accw, acce,
                     di, wbuf, ebuf, di_t, wbuf_t, ebuf_t, zbuf, zebuf):
    c = lax.axis_index("c")
    s = lax.axis_index("s")
    base = (c * NS + s) * EPW

    def zrow(i, carry):
        for j in range(D // 16):
            zbuf[i, pl.ds(j * 16, 16)] = jnp.zeros((16,), _f32)
        zebuf[i, pl.ds(0, 16)] = jnp.zeros((16,), _f32)
        return carry

    lax.fori_loop(0, ZR, zrow, 0)

    def zcopy(j, carry):
        pltpu.sync_copy(zbuf, accw.at[pl.ds(s * RPT + j * ZR, ZR)])
        pltpu.sync_copy(zebuf, acce.at[pl.ds(s * RPT + j * ZR, ZR)])
        return carry

    lax.fori_loop(0, RPT // ZR, zcopy, 0)
    plsc.subcore_barrier()

    def chunk(off, idx, bw, bx, n):
        pltpu.sync_copy(dst_hbm.at[pl.ds(off, n)], idx)
        pltpu.sync_copy(wv_hbm.at[pl.ds(off, n)], bw)
        pltpu.sync_copy(ex_hbm.at[pl.ds(off, n)], bx)
        pltpu.sync_copy(bw, accw.at[idx], add=True)
        pltpu.sync_copy(bx, acce.at[idx], add=True)

    def body(i, carry):
        chunk(base + i * SCH, di, wbuf, ebuf, SCH)
        return carry

    lax.fori_loop(0, SNFULL, body, 0)
    chunk(base + SNFULL * SCH, di_t, wbuf_t, ebuf_t, STAIL)

    plsc.subcore_barrier()
    pltpu.sync_copy(accw.at[pl.ds(s * RPT, RPT)],
                    outw_hbm.at[c, pl.ds(s * RPT, RPT)])
    pltpu.sync_copy(acce.at[pl.ds(s * RPT, RPT)],
                    oute_hbm.at[c, pl.ds(s * RPT, RPT)])


@functools.lru_cache(maxsize=1)
def _sc_kernels():
    mesh = plsc.VectorSubcoreMesh(core_axis_name="c", subcore_axis_name="s")
    gather = functools.partial(
        pl.kernel,
        out_type=(
            jax.ShapeDtypeStruct((EC, KV), _f32),
            jax.ShapeDtypeStruct((EC, D), _f32),
        ),
        mesh=mesh,
        scratch_types=[
            pltpu.VMEM((CH,), jnp.int32),
            pltpu.VMEM((CH,), jnp.int32),
            pltpu.VMEM((CH, KV), _f32),
            pltpu.VMEM((CH, D), _f32),
            pltpu.VMEM((TAIL,), jnp.int32),
            pltpu.VMEM((TAIL,), jnp.int32),
            pltpu.VMEM((TAIL, KV), _f32),
            pltpu.VMEM((TAIL, D), _f32),
            pltpu.SemaphoreType.DMA,
            pltpu.SemaphoreType.DMA,
        ],
    )(_sc_gather_body)
    scatter = functools.partial(
        pl.kernel,
        out_type=(
            jax.ShapeDtypeStruct((NC, NP, D), _f32),
            jax.ShapeDtypeStruct((NC, NP, EW), _f32),
        ),
        mesh=mesh,
        scratch_types=[
            pltpu.VMEM_SHARED((NP, D), _f32),
            pltpu.VMEM_SHARED((NP, EW), _f32),
            pltpu.VMEM((SCH,), jnp.int32),
            pltpu.VMEM((SCH, D), _f32),
            pltpu.VMEM((SCH, EW), _f32),
            pltpu.VMEM((STAIL,), jnp.int32),
            pltpu.VMEM((STAIL, D), _f32),
            pltpu.VMEM((STAIL, EW), _f32),
            pltpu.VMEM((ZR, D), _f32),
            pltpu.VMEM((ZR, EW), _f32),
        ],
        compiler_params=pltpu.CompilerParams(use_tc_tiling_on_sc=False),
    )(_sc_scatter_body)
    return gather, scatter


_BN = 1000
_BE = 512


def _proj(x, wcat, bcat):
    return pl.pallas_call(
        _proj_body,
        grid=(N // _BN,),
        in_specs=[
            pl.BlockSpec((_BN, D), lambda i: (i, 0)),
            pl.BlockSpec((D, 4 * D), lambda i: (0, 0)),
            pl.BlockSpec((1, 4 * D), lambda i: (0, 0)),
        ],
        out_specs=pl.BlockSpec((_BN, 4 * D), lambda i: (i, 0)),
        out_shape=jax.ShapeDtypeStruct((N, 4 * D), _f32),
    )(x, wcat, bcat)


def _edge(qd, kvs, ea_full, we, be, s_mat, st_mat, chunk):
    nb = EC // _BE
    return pl.pallas_call(
        _edge_body,
        grid=(nb,),
        in_specs=[
            pl.BlockSpec((_BE, D), lambda i: (i, 0)),
            pl.BlockSpec((_BE, KV), lambda i: (i, 0)),
            pl.BlockSpec((_BE, 16), lambda i, c=chunk: (c * nb + i, 0)),
            pl.BlockSpec((16, D), lambda i: (0, 0)),
            pl.BlockSpec((1, D), lambda i: (0, 0)),
            pl.BlockSpec((D, H), lambda i: (0, 0)),
            pl.BlockSpec((H, D), lambda i: (0, 0)),
        ],
        out_specs=[
            pl.BlockSpec((_BE, D), lambda i: (i, 0)),
            pl.BlockSpec((_BE, EW), lambda i: (i, 0)),
        ],
        out_shape=(
            jax.ShapeDtypeStruct((EC, D), _f32),
            jax.ShapeDtypeStruct((EC, EW), _f32),
        ),
    )(qd, kvs, ea_full, we, be, s_mat, st_mat)


def _final(parts, x, xs, w1, b1, w2, b2, g1, be1, g2, be2, st_mat):
    return pl.pallas_call(
        _final_body,
        grid=(N // _BN,),
        in_specs=[
            pl.BlockSpec((NC, _BN, D), lambda i: (0, i, 0)),
            pl.BlockSpec((NC, _BN, EW), lambda i: (0, i, 0)),
            pl.BlockSpec((NC, _BN, D), lambda i: (0, i, 0)),
            pl.BlockSpec((NC, _BN, EW), lambda i: (0, i, 0)),
            pl.BlockSpec((NC, _BN, D), lambda i: (0, i, 0)),
            pl.BlockSpec((NC, _BN, EW), lambda i: (0, i, 0)),
            pl.BlockSpec((NC, _BN, D), lambda i: (0, i, 0)),
            pl.BlockSpec((NC, _BN, EW), lambda i: (0, i, 0)),
            pl.BlockSpec((NC, _BN, D), lambda i: (0, i, 0)),
            pl.BlockSpec((NC, _BN, EW), lambda i: (0, i, 0)),
            pl.BlockSpec((_BN, D), lambda i: (i, 0)),
            pl.BlockSpec((_BN, D), lambda i: (i, 0)),
            pl.BlockSpec((D, 4 * D), lambda i: (0, 0)),
            pl.BlockSpec((1, 4 * D), lambda i: (0, 0)),
            pl.BlockSpec((4 * D, D), lambda i: (0, 0)),
            pl.BlockSpec((1, D), lambda i: (0, 0)),
            pl.BlockSpec((1, D), lambda i: (0, 0)),
            pl.BlockSpec((1, D), lambda i: (0, 0)),
            pl.BlockSpec((1, D), lambda i: (0, 0)),
            pl.BlockSpec((1, D), lambda i: (0, 0)),
            pl.BlockSpec((H, D), lambda i: (0, 0)),
        ],
        out_specs=pl.BlockSpec((_BN, D), lambda i: (i, 0)),
        out_shape=jax.ShapeDtypeStruct((N, D), _f32),
    )(*parts, x, xs, w1, b1, w2, b2, g1, be1, g2, be2, st_mat)


def kernel(x, edge_index, edge_attr, Wq, bq, Wk, bk, Wv, bv, We, be,
           Wskip, bskip, g1, be1, W1, bf1, W2, bf2, g2, be2):
    src = edge_index[0]
    dst = edge_index[1]

    wcat = jnp.concatenate([Wk, Wv, Wq, Wskip], axis=1)
    bcat = jnp.concatenate([bk, bv, bq, bskip])[None, :]
    proj = _proj(x, wcat, bcat)
    kv_tab = proj[:, :KV]
    q_tab = proj[:, KV:KV + D]
    xs = proj[:, KV + D:]

    sc_gather, sc_scatter = _sc_kernels()

    hs = jnp.kron(jnp.eye(H, dtype=_f32), jnp.ones((Dh, 1), _f32))
    s_mat = hs * (1.0 / math.sqrt(Dh))
    st_mat = hs.T

    parts = []
    for i in range(NCH):
        sl = slice(i * EC, (i + 1) * EC)
        src_c, dst_c = src[sl], dst[sl]
        kvs, qd = sc_gather(kv_tab, q_tab, src_c, dst_c)
        wv, ex = _edge(qd, kvs, edge_attr, We, be[None, :], s_mat, st_mat, i)
        pw_i, pe_i = sc_scatter(wv, ex, dst_c)
        parts.extend([pw_i, pe_i])

    return _final(parts, x, xs, W1, bf1[None, :], W2, bf2[None, :],
                  g1[None, :], be1[None, :], g2[None, :], be2[None, :], st_mat)


# speedup vs baseline: 35.9872x; 1.0201x over previous
import functools
import math

import jax
import jax.numpy as jnp
from jax import lax
from jax.experimental import pallas as pl
from jax.experimental.pallas import tpu as pltpu
from jax.experimental.pallas import tpu_sc as plsc

N = 10000
E = 320000
D = 128
H = 8
Dh = 16
KV = 2 * D
EW = 16

NCH = 5
EC = E // NCH

NC = 2
NS = 16
NW = NC * NS
EPW = EC // NW
CH = 128
NFULL = EPW // CH
TAIL = EPW - NFULL * CH
NP = 10240
RPT = NP // NS
ZR = 64
SCH = 96
SNFULL = EPW // SCH
STAIL = EPW - SNFULL * SCH

_f32 = jnp.float32


def _proj_body(x_ref, w_ref, b_ref, o_ref):
    o_ref[...] = (
        jnp.dot(x_ref[...], w_ref[...], preferred_element_type=_f32)
        + b_ref[...]
    )


def _edge_body(qd_ref, kvs_ref, ea_ref, we_ref, be_ref, s_ref, st_ref,
               wv_ref, ex_ref):
    e = jnp.dot(ea_ref[...], we_ref[...], preferred_element_type=_f32) + be_ref[...]
    k_e = kvs_ref[:, :D] + e
    v_e = kvs_ref[:, D:] + e
    prod = qd_ref[...] * k_e
    logits = jnp.dot(prod, s_ref[...], preferred_element_type=_f32)
    ex = jnp.exp(logits)
    ex_b = jnp.dot(ex, st_ref[...], preferred_element_type=_f32)
    wv_ref[...] = ex_b * v_e
    ex_ref[:, :H] = ex
    ex_ref[:, H:] = jnp.zeros((ex.shape[0], EW - H), _f32)


def _layernorm(x, g, b, eps=1e-5):
    mu = jnp.mean(x, axis=-1, keepdims=True)
    var = jnp.mean((x - mu) ** 2, axis=-1, keepdims=True)
    return (x - mu) / jnp.sqrt(var + eps) * g + b


def _final_body(pw0, pe0, pw1, pe1, pw2, pe2, pw3, pe3, pw4, pe4, x_ref,
                xs_ref, w1_ref, b1_ref, w2_ref, b2_ref, g1_ref, be1_ref,
                g2_ref, be2_ref, st_ref, o_ref):
    numer = ((pw0[0] + pw0[1]) + (pw1[0] + pw1[1]) + (pw2[0] + pw2[1])
             + (pw3[0] + pw3[1]) + (pw4[0] + pw4[1]))
    eacc = ((pe0[0] + pe0[1]) + (pe1[0] + pe1[1]) + (pe2[0] + pe2[1])
            + (pe3[0] + pe3[1]) + (pe4[0] + pe4[1]))
    den = eacc[:, :H]
    den_b = jnp.dot(jnp.maximum(den, 1e-16), st_ref[...],
                    preferred_element_type=_f32)
    agg = numer / den_b
    conv = agg + xs_ref[...] + x_ref[...]
    h = _layernorm(conv, g1_ref[...], be1_ref[...])
    t = jnp.dot(h, w1_ref[...], preferred_element_type=_f32) + b1_ref[...]
    ff = jnp.dot(jax.nn.gelu(t), w2_ref[...], preferred_element_type=_f32) \
        + b2_ref[...]
    o_ref[...] = _layernorm(ff + h, g2_ref[...], be2_ref[...])


def _sc_gather_body(cb, kv_hbm, q_hbm, src_hbm, dst_hbm, kvs_out, qd_out,
               si, di, kvbuf, qbuf, si_t, di_t, kvbuf_t, qbuf_t, sem_k, sem_q):
    c = lax.axis_index("c")
    s = lax.axis_index("s")
    base = (c * NS + s) * EPW

    def chunk(off, idx_s, idx_d, buf_kv, buf_q, n):
        pltpu.sync_copy(src_hbm.at[pl.ds(cb + off, n)], idx_s)
        pltpu.sync_copy(dst_hbm.at[pl.ds(cb + off, n)], idx_d)
        ck = pltpu.async_copy(kv_hbm.at[idx_s], buf_kv, sem_k)
        cq = pltpu.async_copy(q_hbm.at[idx_d], buf_q, sem_q)
        ck.wait()
        cq.wait()
        pltpu.sync_copy(buf_kv, kvs_out.at[pl.ds(off, n)])
        pltpu.sync_copy(buf_q, qd_out.at[pl.ds(off, n)])

    def body(i, carry):
        chunk(base + i * CH, si, di, kvbuf, qbuf, CH)
        return carry

    lax.fori_loop(0, NFULL, body, 0)
    chunk(base + NFULL * CH, si_t, di_t, kvbuf_t, qbuf_t, TAIL)


def _sc_scatter_body(cb, wv_hbm, ex_hbm, dst_hbm, outw_hbm, oute_hbm, accw,
                     acce, di, wbuf, ebuf, di_t, wbuf_t, ebuf_t, zbuf, zebuf):
    c = lax.axis_index("c")
    s = lax.axis_index("s")
    base = (c * NS + s) * EPW

    def zrow(i, carry):
        for j in range(D // 16):
            zbuf[i, pl.ds(j * 16, 16)] = jnp.zeros((16,), _f32)
        zebuf[i, pl.ds(0, 16)] = jnp.zeros((16,), _f32)
        return carry

    lax.fori_loop(0, ZR, zrow, 0)

    def zcopy(j, carry):
        pltpu.sync_copy(zbuf, accw.at[pl.ds(s * RPT + j * ZR, ZR)])
        pltpu.sync_copy(zebuf, acce.at[pl.ds(s * RPT + j * ZR, ZR)])
        return carry

    lax.fori_loop(0, RPT // ZR, zcopy, 0)
    plsc.subcore_barrier()

    def chunk(off, idx, bw, bx, n):
        pltpu.sync_copy(dst_hbm.at[pl.ds(cb + off, n)], idx)
        pltpu.sync_copy(wv_hbm.at[pl.ds(off, n)], bw)
        pltpu.sync_copy(ex_hbm.at[pl.ds(off, n)], bx)
        pltpu.sync_copy(bw, accw.at[idx], add=True)
        pltpu.sync_copy(bx, acce.at[idx], add=True)

    def body(i, carry):
        chunk(base + i * SCH, di, wbuf, ebuf, SCH)
        return carry

    lax.fori_loop(0, SNFULL, body, 0)
    chunk(base + SNFULL * SCH, di_t, wbuf_t, ebuf_t, STAIL)

    plsc.subcore_barrier()
    pltpu.sync_copy(accw.at[pl.ds(s * RPT, RPT)],
                    outw_hbm.at[c, pl.ds(s * RPT, RPT)])
    pltpu.sync_copy(acce.at[pl.ds(s * RPT, RPT)],
                    oute_hbm.at[c, pl.ds(s * RPT, RPT)])


@functools.lru_cache(maxsize=NCH)
def _sc_kernels(chunk):
    cb = chunk * EC
    mesh = plsc.VectorSubcoreMesh(core_axis_name="c", subcore_axis_name="s")
    gather = functools.partial(
        pl.kernel,
        out_type=(
            jax.ShapeDtypeStruct((EC, KV), _f32),
            jax.ShapeDtypeStruct((EC, D), _f32),
        ),
        mesh=mesh,
        scratch_types=[
            pltpu.VMEM((CH,), jnp.int32),
            pltpu.VMEM((CH,), jnp.int32),
            pltpu.VMEM((CH, KV), _f32),
            pltpu.VMEM((CH, D), _f32),
            pltpu.VMEM((TAIL,), jnp.int32),
            pltpu.VMEM((TAIL,), jnp.int32),
            pltpu.VMEM((TAIL, KV), _f32),
            pltpu.VMEM((TAIL, D), _f32),
            pltpu.SemaphoreType.DMA,
            pltpu.SemaphoreType.DMA,
        ],
    )(functools.partial(_sc_gather_body, cb))
    scatter = functools.partial(
        pl.kernel,
        out_type=(
            jax.ShapeDtypeStruct((NC, NP, D), _f32),
            jax.ShapeDtypeStruct((NC, NP, EW), _f32),
        ),
        mesh=mesh,
        scratch_types=[
            pltpu.VMEM_SHARED((NP, D), _f32),
            pltpu.VMEM_SHARED((NP, EW), _f32),
            pltpu.VMEM((SCH,), jnp.int32),
            pltpu.VMEM((SCH, D), _f32),
            pltpu.VMEM((SCH, EW), _f32),
            pltpu.VMEM((STAIL,), jnp.int32),
            pltpu.VMEM((STAIL, D), _f32),
            pltpu.VMEM((STAIL, EW), _f32),
            pltpu.VMEM((ZR, D), _f32),
            pltpu.VMEM((ZR, EW), _f32),
        ],
        compiler_params=pltpu.CompilerParams(use_tc_tiling_on_sc=False),
    )(functools.partial(_sc_scatter_body, cb))
    return gather, scatter


_BN = 1000
_BE = 512


def _proj(x, wcat, bcat):
    return pl.pallas_call(
        _proj_body,
        grid=(N // _BN,),
        in_specs=[
            pl.BlockSpec((_BN, D), lambda i: (i, 0)),
            pl.BlockSpec((D, 4 * D), lambda i: (0, 0)),
            pl.BlockSpec((1, 4 * D), lambda i: (0, 0)),
        ],
        out_specs=pl.BlockSpec((_BN, 4 * D), lambda i: (i, 0)),
        out_shape=jax.ShapeDtypeStruct((N, 4 * D), _f32),
    )(x, wcat, bcat)


def _edge(qd, kvs, ea_full, we, be, s_mat, st_mat, chunk):
    nb = EC // _BE
    return pl.pallas_call(
        _edge_body,
        grid=(nb,),
        in_specs=[
            pl.BlockSpec((_BE, D), lambda i: (i, 0)),
            pl.BlockSpec((_BE, KV), lambda i: (i, 0)),
            pl.BlockSpec((_BE, 16), lambda i, c=chunk: (c * nb + i, 0)),
            pl.BlockSpec((16, D), lambda i: (0, 0)),
            pl.BlockSpec((1, D), lambda i: (0, 0)),
            pl.BlockSpec((D, H), lambda i: (0, 0)),
            pl.BlockSpec((H, D), lambda i: (0, 0)),
        ],
        out_specs=[
            pl.BlockSpec((_BE, D), lambda i: (i, 0)),
            pl.BlockSpec((_BE, EW), lambda i: (i, 0)),
        ],
        out_shape=(
            jax.ShapeDtypeStruct((EC, D), _f32),
            jax.ShapeDtypeStruct((EC, EW), _f32),
        ),
    )(qd, kvs, ea_full, we, be, s_mat, st_mat)


def _final(parts, x, xs, w1, b1, w2, b2, g1, be1, g2, be2, st_mat):
    return pl.pallas_call(
        _final_body,
        grid=(N // _BN,),
        in_specs=[
            pl.BlockSpec((NC, _BN, D), lambda i: (0, i, 0)),
            pl.BlockSpec((NC, _BN, EW), lambda i: (0, i, 0)),
            pl.BlockSpec((NC, _BN, D), lambda i: (0, i, 0)),
            pl.BlockSpec((NC, _BN, EW), lambda i: (0, i, 0)),
            pl.BlockSpec((NC, _BN, D), lambda i: (0, i, 0)),
            pl.BlockSpec((NC, _BN, EW), lambda i: (0, i, 0)),
            pl.BlockSpec((NC, _BN, D), lambda i: (0, i, 0)),
            pl.BlockSpec((NC, _BN, EW), lambda i: (0, i, 0)),
            pl.BlockSpec((NC, _BN, D), lambda i: (0, i, 0)),
            pl.BlockSpec((NC, _BN, EW), lambda i: (0, i, 0)),
            pl.BlockSpec((_BN, D), lambda i: (i, 0)),
            pl.BlockSpec((_BN, D), lambda i: (i, 0)),
            pl.BlockSpec((D, 4 * D), lambda i: (0, 0)),
            pl.BlockSpec((1, 4 * D), lambda i: (0, 0)),
            pl.BlockSpec((4 * D, D), lambda i: (0, 0)),
            pl.BlockSpec((1, D), lambda i: (0, 0)),
            pl.BlockSpec((1, D), lambda i: (0, 0)),
            pl.BlockSpec((1, D), lambda i: (0, 0)),
            pl.BlockSpec((1, D), lambda i: (0, 0)),
            pl.BlockSpec((1, D), lambda i: (0, 0)),
            pl.BlockSpec((H, D), lambda i: (0, 0)),
        ],
        out_specs=pl.BlockSpec((_BN, D), lambda i: (i, 0)),
        out_shape=jax.ShapeDtypeStruct((N, D), _f32),
    )(*parts, x, xs, w1, b1, w2, b2, g1, be1, g2, be2, st_mat)


def kernel(x, edge_index, edge_attr, Wq, bq, Wk, bk, Wv, bv, We, be,
           Wskip, bskip, g1, be1, W1, bf1, W2, bf2, g2, be2):
    src = edge_index[0]
    dst = edge_index[1]

    wcat = jnp.concatenate([Wk, Wv, Wq, Wskip], axis=1)
    bcat = jnp.concatenate([bk, bv, bq, bskip])[None, :]
    proj = _proj(x, wcat, bcat)
    kv_tab = proj[:, :KV]
    q_tab = proj[:, KV:KV + D]
    xs = proj[:, KV + D:]

    hs = jnp.kron(jnp.eye(H, dtype=_f32), jnp.ones((Dh, 1), _f32))
    s_mat = hs * (1.0 / math.sqrt(Dh))
    st_mat = hs.T

    parts = []
    for i in range(NCH):
        sc_gather, sc_scatter = _sc_kernels(i)
        kvs, qd = sc_gather(kv_tab, q_tab, src, dst)
        wv, ex = _edge(qd, kvs, edge_attr, We, be[None, :], s_mat, st_mat, i)
        pw_i, pe_i = sc_scatter(wv, ex, dst)
        parts.extend([pw_i, pe_i])

    return _final(parts, x, xs, W1, bf1[None, :], W2, bf2[None, :],
                  g1[None, :], be1[None, :], g2[None, :], be2[None, :], st_mat)


# speedup vs baseline: 41.9465x; 1.1656x over previous
import functools
import math

import jax
import jax.numpy as jnp
from jax import lax
from jax.experimental import pallas as pl
from jax.experimental.pallas import tpu as pltpu
from jax.experimental.pallas import tpu_sc as plsc

N = 10000
E = 320000
D = 128
H = 8
Dh = 16
KV = 2 * D
EW = 16

NCH = 5
EC = E // NCH

NC = 2
NS = 16
NW = NC * NS
EPW = EC // NW
CH = 128
NFULL = EPW // CH
TAIL = EPW - NFULL * CH
NP = 10240
RPT = NP // NS
ZR = 64
SCH = 96
SNFULL = EPW // SCH
STAIL = EPW - SNFULL * SCH

_f32 = jnp.float32


def _proj_body(x_ref, w_ref, b_ref, kv_ref, q_ref, xs_ref):
    t = (
        jnp.dot(x_ref[...], w_ref[...], preferred_element_type=_f32)
        + b_ref[...]
    )
    kv_ref[...] = t[:, :KV]
    q_ref[...] = t[:, KV:KV + D]
    xs_ref[...] = t[:, KV + D:]


def _edge_body(qd_ref, kvs_ref, eat_ref, we_ref, be_ref, s_ref, st_ref,
               wv_ref, ex_ref):
    e = lax.dot_general(
        eat_ref[...], we_ref[...], (((0,), (0,)), ((), ())),
        preferred_element_type=_f32,
    ) + be_ref[...]
    k_e = kvs_ref[:, :D] + e
    v_e = kvs_ref[:, D:] + e
    prod = qd_ref[...] * k_e
    logits = jnp.dot(prod, s_ref[...], preferred_element_type=_f32)
    ex = jnp.exp(logits)
    ex_b = jnp.dot(ex, st_ref[...], preferred_element_type=_f32)
    wv_ref[...] = ex_b * v_e
    ex_ref[:, :H] = ex
    ex_ref[:, H:] = jnp.zeros((ex.shape[0], D - H), _f32)


def _layernorm(x, g, b, eps=1e-5):
    mu = jnp.mean(x, axis=-1, keepdims=True)
    var = jnp.mean((x - mu) ** 2, axis=-1, keepdims=True)
    return (x - mu) / jnp.sqrt(var + eps) * g + b


def _final_body(pw0, pe0, pw1, pe1, pw2, pe2, pw3, pe3, pw4, pe4, x_ref,
                xs_ref, w1_ref, b1_ref, w2_ref, b2_ref, g1_ref, be1_ref,
                g2_ref, be2_ref, st_ref, o_ref):
    numer = ((pw0[0] + pw0[1]) + (pw1[0] + pw1[1]) + (pw2[0] + pw2[1])
             + (pw3[0] + pw3[1]) + (pw4[0] + pw4[1]))
    eacc = ((pe0[0] + pe0[1]) + (pe1[0] + pe1[1]) + (pe2[0] + pe2[1])
            + (pe3[0] + pe3[1]) + (pe4[0] + pe4[1]))
    den = eacc[:, :H]
    den_b = jnp.dot(jnp.maximum(den, 1e-16), st_ref[...],
                    preferred_element_type=_f32)
    agg = numer / den_b
    conv = agg + xs_ref[...] + x_ref[...]
    h = _layernorm(conv, g1_ref[...], be1_ref[...])
    t = jnp.dot(h, w1_ref[...], preferred_element_type=_f32) + b1_ref[...]
    ff = jnp.dot(jax.nn.gelu(t), w2_ref[...], preferred_element_type=_f32) \
        + b2_ref[...]
    o_ref[...] = _layernorm(ff + h, g2_ref[...], be2_ref[...])


def _sc_gather_body(cb, kv_hbm, q_hbm, src_hbm, dst_hbm, kvs_out, qd_out,
               si, di, kvbuf, qbuf, si_t, di_t, kvbuf_t, qbuf_t, sem_k, sem_q):
    c = lax.axis_index("c")
    s = lax.axis_index("s")
    base = (c * NS + s) * EPW

    def chunk(off, idx_s, idx_d, buf_kv, buf_q, n):
        pltpu.sync_copy(src_hbm.at[pl.ds(cb + off, n)], idx_s)
        pltpu.sync_copy(dst_hbm.at[pl.ds(cb + off, n)], idx_d)
        ck = pltpu.async_copy(kv_hbm.at[idx_s], buf_kv, sem_k)
        cq = pltpu.async_copy(q_hbm.at[idx_d], buf_q, sem_q)
        ck.wait()
        cq.wait()
        pltpu.sync_copy(buf_kv, kvs_out.at[pl.ds(off, n)])
        pltpu.sync_copy(buf_q, qd_out.at[pl.ds(off, n)])

    def body(i, carry):
        chunk(base + i * CH, si, di, kvbuf, qbuf, CH)
        return carry

    lax.fori_loop(0, NFULL, body, 0)
    chunk(base + NFULL * CH, si_t, di_t, kvbuf_t, qbuf_t, TAIL)


def _sc_scatter_body(cb, wv_hbm, ex_hbm, dst_hbm, outw_hbm, oute_hbm, accw,
                     acce, di, wbuf, ebuf, di_t, wbuf_t, ebuf_t, zbuf, zebuf):
    c = lax.axis_index("c")
    s = lax.axis_index("s")
    base = (c * NS + s) * EPW

    def zrow(i, carry):
        for j in range(D // 16):
            zbuf[i, pl.ds(j * 16, 16)] = jnp.zeros((16,), _f32)
        zebuf[i, pl.ds(0, 16)] = jnp.zeros((16,), _f32)
        return carry

    lax.fori_loop(0, ZR, zrow, 0)

    def zcopy(j, carry):
        pltpu.sync_copy(zbuf, accw.at[pl.ds(s * RPT + j * ZR, ZR)])
        pltpu.sync_copy(zebuf, acce.at[pl.ds(s * RPT + j * ZR, ZR)])
        return carry

    lax.fori_loop(0, RPT // ZR, zcopy, 0)
    plsc.subcore_barrier()

    def chunk(off, idx, bw, bx, n):
        pltpu.sync_copy(dst_hbm.at[pl.ds(cb + off, n)], idx)
        pltpu.sync_copy(wv_hbm.at[pl.ds(off, n)], bw)
        pltpu.sync_copy(ex_hbm.at[pl.ds(off, n), pl.ds(0, EW)], bx)
        pltpu.sync_copy(bw, accw.at[idx], add=True)
        pltpu.sync_copy(bx, acce.at[idx], add=True)

    def body(i, carry):
        chunk(base + i * SCH, di, wbuf, ebuf, SCH)
        return carry

    lax.fori_loop(0, SNFULL, body, 0)
    chunk(base + SNFULL * SCH, di_t, wbuf_t, ebuf_t, STAIL)

    plsc.subcore_barrier()
    pltpu.sync_copy(accw.at[pl.ds(s * RPT, RPT)],
                    outw_hbm.at[c, pl.ds(s * RPT, RPT)])
    pltpu.sync_copy(acce.at[pl.ds(s * RPT, RPT)],
                    oute_hbm.at[c, pl.ds(s * RPT, RPT)])


@functools.lru_cache(maxsize=NCH)
def _sc_kernels(chunk):
    cb = chunk * EC
    mesh = plsc.VectorSubcoreMesh(core_axis_name="c", subcore_axis_name="s")
    gather = functools.partial(
        pl.kernel,
        out_type=(
            jax.ShapeDtypeStruct((EC, KV), _f32),
            jax.ShapeDtypeStruct((EC, D), _f32),
        ),
        mesh=mesh,
        scratch_types=[
            pltpu.VMEM((CH,), jnp.int32),
            pltpu.VMEM((CH,), jnp.int32),
            pltpu.VMEM((CH, KV), _f32),
            pltpu.VMEM((CH, D), _f32),
            pltpu.VMEM((TAIL,), jnp.int32),
            pltpu.VMEM((TAIL,), jnp.int32),
            pltpu.VMEM((TAIL, KV), _f32),
            pltpu.VMEM((TAIL, D), _f32),
            pltpu.SemaphoreType.DMA,
            pltpu.SemaphoreType.DMA,
        ],
    )(functools.partial(_sc_gather_body, cb))
    scatter = functools.partial(
        pl.kernel,
        out_type=(
            jax.ShapeDtypeStruct((NC, NP, D), _f32),
            jax.ShapeDtypeStruct((NC, NP, EW), _f32),
        ),
        mesh=mesh,
        scratch_types=[
            pltpu.VMEM_SHARED((NP, D), _f32),
            pltpu.VMEM_SHARED((NP, EW), _f32),
            pltpu.VMEM((SCH,), jnp.int32),
            pltpu.VMEM((SCH, D), _f32),
            pltpu.VMEM((SCH, EW), _f32),
            pltpu.VMEM((STAIL,), jnp.int32),
            pltpu.VMEM((STAIL, D), _f32),
            pltpu.VMEM((STAIL, EW), _f32),
            pltpu.VMEM((ZR, D), _f32),
            pltpu.VMEM((ZR, EW), _f32),
        ],
        compiler_params=pltpu.CompilerParams(use_tc_tiling_on_sc=False),
    )(functools.partial(_sc_scatter_body, cb))
    return gather, scatter


_BN = 1000
_BE = 512


def _proj(x, wcat, bcat):
    return pl.pallas_call(
        _proj_body,
        grid=(N // _BN,),
        in_specs=[
            pl.BlockSpec((_BN, D), lambda i: (i, 0)),
            pl.BlockSpec((D, 4 * D), lambda i: (0, 0)),
            pl.BlockSpec((1, 4 * D), lambda i: (0, 0)),
        ],
        out_specs=[
            pl.BlockSpec((_BN, KV), lambda i: (i, 0)),
            pl.BlockSpec((_BN, D), lambda i: (i, 0)),
            pl.BlockSpec((_BN, D), lambda i: (i, 0)),
        ],
        out_shape=(
            jax.ShapeDtypeStruct((N, KV), _f32),
            jax.ShapeDtypeStruct((N, D), _f32),
            jax.ShapeDtypeStruct((N, D), _f32),
        ),
    )(x, wcat, bcat)


def _edge(qd, kvs, ea_t, we, be, s_mat, st_mat, chunk):
    nb = EC // _BE
    return pl.pallas_call(
        _edge_body,
        grid=(nb,),
        in_specs=[
            pl.BlockSpec((_BE, D), lambda i: (i, 0)),
            pl.BlockSpec((_BE, KV), lambda i: (i, 0)),
            pl.BlockSpec((16, _BE), lambda i, c=chunk: (0, c * nb + i)),
            pl.BlockSpec((16, D), lambda i: (0, 0)),
            pl.BlockSpec((1, D), lambda i: (0, 0)),
            pl.BlockSpec((D, H), lambda i: (0, 0)),
            pl.BlockSpec((H, D), lambda i: (0, 0)),
        ],
        out_specs=[
            pl.BlockSpec((_BE, D), lambda i: (i, 0)),
            pl.BlockSpec((_BE, D), lambda i: (i, 0)),
        ],
        out_shape=(
            jax.ShapeDtypeStruct((EC, D), _f32),
            jax.ShapeDtypeStruct((EC, D), _f32),
        ),
    )(qd, kvs, ea_t, we, be, s_mat, st_mat)


def _final(parts, x, xs, w1, b1, w2, b2, g1, be1, g2, be2, st_mat):
    return pl.pallas_call(
        _final_body,
        grid=(N // _BN,),
        in_specs=[
            pl.BlockSpec((NC, _BN, D), lambda i: (0, i, 0)),
            pl.BlockSpec((NC, _BN, EW), lambda i: (0, i, 0)),
            pl.BlockSpec((NC, _BN, D), lambda i: (0, i, 0)),
            pl.BlockSpec((NC, _BN, EW), lambda i: (0, i, 0)),
            pl.BlockSpec((NC, _BN, D), lambda i: (0, i, 0)),
            pl.BlockSpec((NC, _BN, EW), lambda i: (0, i, 0)),
            pl.BlockSpec((NC, _BN, D), lambda i: (0, i, 0)),
            pl.BlockSpec((NC, _BN, EW), lambda i: (0, i, 0)),
            pl.BlockSpec((NC, _BN, D), lambda i: (0, i, 0)),
            pl.BlockSpec((NC, _BN, EW), lambda i: (0, i, 0)),
            pl.BlockSpec((_BN, D), lambda i: (i, 0)),
            pl.BlockSpec((_BN, D), lambda i: (i, 0)),
            pl.BlockSpec((D, 4 * D), lambda i: (0, 0)),
            pl.BlockSpec((1, 4 * D), lambda i: (0, 0)),
            pl.BlockSpec((4 * D, D), lambda i: (0, 0)),
            pl.BlockSpec((1, D), lambda i: (0, 0)),
            pl.BlockSpec((1, D), lambda i: (0, 0)),
            pl.BlockSpec((1, D), lambda i: (0, 0)),
            pl.BlockSpec((1, D), lambda i: (0, 0)),
            pl.BlockSpec((1, D), lambda i: (0, 0)),
            pl.BlockSpec((H, D), lambda i: (0, 0)),
        ],
        out_specs=pl.BlockSpec((_BN, D), lambda i: (i, 0)),
        out_shape=jax.ShapeDtypeStruct((N, D), _f32),
    )(*parts, x, xs, w1, b1, w2, b2, g1, be1, g2, be2, st_mat)


def kernel(x, edge_index, edge_attr, Wq, bq, Wk, bk, Wv, bv, We, be,
           Wskip, bskip, g1, be1, W1, bf1, W2, bf2, g2, be2):
    src = edge_index[0]
    dst = edge_index[1]

    wcat = jnp.concatenate([Wk, Wv, Wq, Wskip], axis=1)
    bcat = jnp.concatenate([bk, bv, bq, bskip])[None, :]
    kv_tab, q_tab, xs = _proj(x, wcat, bcat)
    ea_t = edge_attr.T

    hs = jnp.kron(jnp.eye(H, dtype=_f32), jnp.ones((Dh, 1), _f32))
    s_mat = hs * (1.0 / math.sqrt(Dh))
    st_mat = hs.T

    parts = []
    for i in range(NCH):
        sc_gather, sc_scatter = _sc_kernels(i)
        kvs, qd = sc_gather(kv_tab, q_tab, src, dst)
        wv, ex = _edge(qd, kvs, ea_t, We, be[None, :], s_mat, st_mat, i)
        pw_i, pe_i = sc_scatter(wv, ex, dst)
        parts.extend([pw_i, pe_i])

    return _final(parts, x, xs, W1, bf1[None, :], W2, bf2[None, :],
                  g1[None, :], be1[None, :], g2[None, :], be2[None, :], st_mat)
